# baseline (speedup 1.0000x reference)
import jax
import jax.numpy as jnp
from jax.experimental import pallas as pl
from jax.experimental.pallas import tpu as pltpu

_N = 1000
_F = 26
_W = 512
_TB = 128
_NCOLS = _F * _N


def _onehot_body(x_ref, o_ref):
    xb = x_ref[...]
    col = jax.lax.broadcasted_iota(jnp.int32, (_TB, _W), 1)
    for c in range(-(-_NCOLS // _W)):
        base = c * _W
        w = min(_W, _NCOLS - base)
        f0 = base // _N
        f1 = min((base + w - 1) // _N, _F - 1)
        m = (col + (base - f0 * _N)) == xb[:, f0:f0 + 1]
        if f1 != f0:
            m = m | ((col + (base - f1 * _N)) == xb[:, f1:f1 + 1])
        o_ref[:, base:base + w] = m[:, :w].astype(jnp.float32)


def kernel(x):
    if x.ndim == 1:
        x = x[None, :]
    b, f = x.shape
    assert f == _F
    nb = -(-b // _TB)

    if b % _TB:
        x = jnp.pad(x, ((0, nb * _TB - b), (0, 0)))

    out = pl.pallas_call(
        _onehot_body,
        grid=(nb,),
        in_specs=[pl.BlockSpec((_TB, _F), lambda bb: (bb, 0))],
        out_specs=pl.BlockSpec((_TB, _NCOLS), lambda bb: (bb, 0)),
        out_shape=jax.ShapeDtypeStruct((nb * _TB, _NCOLS), jnp.float32),
        compiler_params=pltpu.CompilerParams(
            dimension_semantics=("arbitrary",),
        ),
    )(x)

    return out[:b]
